# baseline (speedup 1.0000x reference)
import jax
import jax.numpy as jnp
from jax import lax
from jax.experimental import pallas as pl
from jax.experimental.pallas import tpu as pltpu
from jax.experimental.pallas import tpu_sc as plsc

N = 10000
E = 320000
D = 128
NUM_GNNS = 4
S = 3

NSC = 2
NSUB = 16
NT = NSC * NSUB
EPT = E // NT
CB = 128
NCH = 80
PADE = NCH * CB - EPT
NPAD = N + PADE
SPAD = 640 * NSUB

_MESH = dict(core_axis_name="c", subcore_axis_name="s",
             num_cores=NSC, num_subcores=NSUB)
_CPARAMS = pltpu.CompilerParams(needs_layout_passes=False)


def _scale_rows(gb, wst):
    def row(r, rc):
        wv = plsc.load_gather(wst, [jnp.full((16,), r, jnp.int32)])
        for k in range(D // 16):
            ds = pl.ds(k * 16, 16)
            gb[r, ds] = gb[r, ds] * wv
        return rc

    lax.fori_loop(0, CB, row, 0, unroll=16)


def _unpack_idx(cb16, c, gst, sst):
    for k in range(CB // 16):
        ds = pl.ds(k * 16, 16)
        v = cb16[c, ds]
        gst[ds] = v & jnp.int32(0xFFFF)
        sst[ds] = lax.shift_right_logical(v, 16)


def _compute_w(c, elst, erst, wst):
    lane = lax.iota(jnp.int32, 16)
    for v in range(CB // 16):
        ds = pl.ds(v * 16, 16)
        t = elst[ds] + erst[ds]
        w = jnp.exp(jnp.where(t > 0, t, 0.2 * t))
        pos = c * CB + v * 16 + lane
        wst[ds] = jnp.where(pos < EPT, w, 0.0)


def _edge_rows_body(z_hbm, el_hbm, er_hbm, cb16_hbm, zrows_hbm, zs_hbm,
                    outp_hbm, sp_hbm,
                    acc, sacc, cb16, gst0, gst1, sst0, sst1, wst0, wst1,
                    elst0, elst1, erst0, erst1, gb0, gb1,
                    sg0, sg1, ss0, ss1, se0, se1, sw0, sw1):
    cid = lax.axis_index("c")
    sid = lax.axis_index("s")
    wid = cid * NSUB + sid

    pltpu.sync_copy(cb16_hbm.at[wid], cb16)

    @pl.when(sid == 0)
    def _():
        pltpu.sync_copy(zrows_hbm, acc)
        pltpu.sync_copy(zs_hbm, sacc)
    plsc.subcore_barrier()

    def launch(c, gst, sst, gb, sg, elst, erst, se):
        _unpack_idx(cb16, c, gst, sst)
        pltpu.async_copy(z_hbm.at[gst], gb, sg)
        pltpu.async_copy(el_hbm.at[gst], elst, se)
        pltpu.async_copy(er_hbm.at[sst], erst, se)

    def process(c, gst, sst, gb, sg, elst, erst, se, wst, sw):
        pltpu.make_async_copy(z_hbm.at[gst], gb, sg).wait()
        pltpu.make_async_copy(el_hbm.at[gst], elst, se).wait()
        pltpu.make_async_copy(er_hbm.at[sst], erst, se).wait()
        _compute_w(c, elst, erst, wst)
        pltpu.make_async_copy(wst, sacc.at[sst], sw).start(add=True)
        _scale_rows(gb, wst)

    launch(0, gst0, sst0, gb0, sg0, elst0, erst0, se0)

    def pair(i, carry):
        c0 = 2 * i
        c1 = c0 + 1

        @pl.when(i > 0)
        def _():
            pltpu.make_async_copy(gb1, acc.at[sst1], ss1).wait()
            pltpu.make_async_copy(wst1, sacc.at[sst1], sw1).wait()
        launch(c1, gst1, sst1, gb1, sg1, elst1, erst1, se1)

        process(c0, gst0, sst0, gb0, sg0, elst0, erst0, se0, wst0, sw0)
        pltpu.make_async_copy(gb0, acc.at[sst0], ss0).start(add=True)

        process(c1, gst1, sst1, gb1, sg1, elst1, erst1, se1, wst1, sw1)

        pltpu.make_async_copy(gb0, acc.at[sst0], ss0).wait()
        pltpu.make_async_copy(wst0, sacc.at[sst0], sw0).wait()

        @pl.when(c0 + 2 < NCH)
        def _():
            launch(c0 + 2, gst0, sst0, gb0, sg0, elst0, erst0, se0)

        pltpu.make_async_copy(gb1, acc.at[sst1], ss1).start(add=True)
        return carry

    lax.fori_loop(0, NCH // 2, pair, 0)
    pltpu.make_async_copy(gb1, acc.at[sst1], ss1).wait()
    pltpu.make_async_copy(wst1, sacc.at[sst1], sw1).wait()

    plsc.subcore_barrier()
    pltpu.sync_copy(sacc.at[pl.ds(sid * 640, 640)],
                    sp_hbm.at[cid, pl.ds(sid * 640, 640)])
    base = sid * 624
    pltpu.sync_copy(acc.at[pl.ds(base, 624)],
                    outp_hbm.at[cid, pl.ds(base, 624)])

    @pl.when(sid == NSUB - 1)
    def _():
        pltpu.sync_copy(acc.at[pl.ds(16 * 624, N - 16 * 624)],
                        outp_hbm.at[cid, pl.ds(16 * 624, N - 16 * 624)])


def _prop_body(z_hbm, cb16_hbm, zrows_hbm,
               outp_hbm,
               acc, cb16, gst0, gst1, sst0, sst1, gb0, gb1,
               sg0, sg1, ss0, ss1):
    cid = lax.axis_index("c")
    sid = lax.axis_index("s")
    wid = cid * NSUB + sid

    pltpu.sync_copy(cb16_hbm.at[wid], cb16)

    @pl.when(sid == 0)
    def _():
        pltpu.sync_copy(zrows_hbm, acc.at[pl.ds(0, N)])
    plsc.subcore_barrier()

    _unpack_idx(cb16, 0, gst0, sst0)
    pltpu.async_copy(z_hbm.at[gst0], gb0, sg0)

    def pair(i, carry):
        c0 = 2 * i
        c1 = c0 + 1

        @pl.when(i > 0)
        def _():
            pltpu.make_async_copy(gb1, acc.at[sst1], ss1).wait()
        _unpack_idx(cb16, c1, gst1, sst1)
        pltpu.async_copy(z_hbm.at[gst1], gb1, sg1)

        pltpu.make_async_copy(z_hbm.at[gst0], gb0, sg0).wait()
        pltpu.make_async_copy(gb0, acc.at[sst0], ss0).start(add=True)

        pltpu.make_async_copy(z_hbm.at[gst1], gb1, sg1).wait()
        pltpu.make_async_copy(gb0, acc.at[sst0], ss0).wait()

        @pl.when(c0 + 2 < NCH)
        def _():
            _unpack_idx(cb16, c0 + 2, gst0, sst0)
            pltpu.async_copy(z_hbm.at[gst0], gb0, sg0)

        pltpu.make_async_copy(gb1, acc.at[sst1], ss1).start(add=True)
        return carry

    lax.fori_loop(0, NCH // 2, pair, 0)
    pltpu.make_async_copy(gb1, acc.at[sst1], ss1).wait()

    plsc.subcore_barrier()
    base = sid * 624
    pltpu.sync_copy(acc.at[pl.ds(base, 624)],
                    outp_hbm.at[cid, pl.ds(base, 624)])

    @pl.when(sid == NSUB - 1)
    def _():
        pltpu.sync_copy(acc.at[pl.ds(16 * 624, N - 16 * 624)],
                        outp_hbm.at[cid, pl.ds(16 * 624, N - 16 * 624)])


def _sc_edge_rows(z, el, er, cb16g, zrows, zs):
    f = pl.kernel(
        _edge_rows_body,
        out_type=(jax.ShapeDtypeStruct((NSC, N, D), jnp.float32),
                  jax.ShapeDtypeStruct((NSC, SPAD), jnp.float32)),
        mesh=plsc.VectorSubcoreMesh(**_MESH),
        compiler_params=_CPARAMS,
        scratch_types=[
            pltpu.VMEM_SHARED((N, D), jnp.float32),
            pltpu.VMEM_SHARED((SPAD,), jnp.float32),
            pltpu.VMEM((NCH, CB), jnp.int32),
            pltpu.VMEM((CB,), jnp.int32),
            pltpu.VMEM((CB,), jnp.int32),
            pltpu.VMEM((CB,), jnp.int32),
            pltpu.VMEM((CB,), jnp.int32),
            pltpu.VMEM((CB,), jnp.float32),
            pltpu.VMEM((CB,), jnp.float32),
            pltpu.VMEM((CB,), jnp.float32),
            pltpu.VMEM((CB,), jnp.float32),
            pltpu.VMEM((CB,), jnp.float32),
            pltpu.VMEM((CB,), jnp.float32),
            pltpu.VMEM((CB, D), jnp.float32),
            pltpu.VMEM((CB, D), jnp.float32),
            pltpu.SemaphoreType.DMA,
            pltpu.SemaphoreType.DMA,
            pltpu.SemaphoreType.DMA,
            pltpu.SemaphoreType.DMA,
            pltpu.SemaphoreType.DMA,
            pltpu.SemaphoreType.DMA,
            pltpu.SemaphoreType.DMA,
            pltpu.SemaphoreType.DMA,
        ],
    )
    return f(z, el, er, cb16g, zrows, zs)


def _sc_prop(zp, cb16p, zrows):
    f = pl.kernel(
        _prop_body,
        out_type=jax.ShapeDtypeStruct((NSC, N, D), jnp.float32),
        mesh=plsc.VectorSubcoreMesh(**_MESH),
        compiler_params=_CPARAMS,
        scratch_types=[
            pltpu.VMEM_SHARED((NPAD, D), jnp.float32),
            pltpu.VMEM((NCH, CB), jnp.int32),
            pltpu.VMEM((CB,), jnp.int32),
            pltpu.VMEM((CB,), jnp.int32),
            pltpu.VMEM((CB,), jnp.int32),
            pltpu.VMEM((CB,), jnp.int32),
            pltpu.VMEM((CB, D), jnp.float32),
            pltpu.VMEM((CB, D), jnp.float32),
            pltpu.SemaphoreType.DMA,
            pltpu.SemaphoreType.DMA,
            pltpu.SemaphoreType.DMA,
            pltpu.SemaphoreType.DMA,
        ],
    )
    return f(zp, cb16p, zrows)


_GRID = 10
_BN = N // _GRID


def _pre_kernel(x_ref, w_ref, b_ref, o_ref):
    o_ref[...] = (jnp.dot(x_ref[...], w_ref[...],
                          preferred_element_type=jnp.float32) + b_ref[...])


def _tc_pre(x, fc_W, fc_b):
    return pl.pallas_call(
        _pre_kernel,
        grid=(_GRID,),
        in_specs=[
            pl.BlockSpec((_BN, D), lambda i: (i, 0)),
            pl.BlockSpec((D, D), lambda i: (0, 0)),
            pl.BlockSpec((1, D), lambda i: (0, 0)),
        ],
        out_specs=pl.BlockSpec((_BN, D), lambda i: (i, 0)),
        out_shape=jax.ShapeDtypeStruct((N, D), jnp.float32),
    )(x, fc_W, fc_b)


def _zelr_kernel(h_ref, w_ref, al_ref, ar_ref, z_ref, el_ref, er_ref):
    z = jnp.dot(h_ref[...], w_ref[...], preferred_element_type=jnp.float32)
    z_ref[...] = z
    el_ref[...] = jnp.dot(z, al_ref[...], preferred_element_type=jnp.float32)
    er_ref[...] = jnp.dot(z, ar_ref[...], preferred_element_type=jnp.float32)


def _tc_zelr(h, W, al, ar):
    return pl.pallas_call(
        _zelr_kernel,
        grid=(_GRID,),
        in_specs=[
            pl.BlockSpec((_BN, D), lambda i: (i, 0)),
            pl.BlockSpec((D, D), lambda i: (0, 0)),
            pl.BlockSpec((D, 1), lambda i: (0, 0)),
            pl.BlockSpec((D, 1), lambda i: (0, 0)),
        ],
        out_specs=[
            pl.BlockSpec((_BN, D), lambda i: (i, 0)),
            pl.BlockSpec((_BN, 1), lambda i: (i, 0)),
            pl.BlockSpec((_BN, 1), lambda i: (i, 0)),
        ],
        out_shape=[
            jax.ShapeDtypeStruct((N, D), jnp.float32),
            jax.ShapeDtypeStruct((N, 1), jnp.float32),
            jax.ShapeDtypeStruct((N, 1), jnp.float32),
        ],
    )(h, W, al, ar)


def _finish_kernel(p_ref, s_ref, b_ref, h_ref):
    ps = p_ref[0] + p_ref[1]
    ss = s_ref[0] + s_ref[1]
    t = ps / (ss + 1e-9) + b_ref[...]
    h_ref[...] = jnp.where(t > 0, t, jnp.exp(jnp.minimum(t, 0.0)) - 1.0)


def _tc_finish(p, s3, b):
    return pl.pallas_call(
        _finish_kernel,
        grid=(_GRID,),
        in_specs=[
            pl.BlockSpec((NSC, _BN, D), lambda i: (0, i, 0)),
            pl.BlockSpec((NSC, _BN, 1), lambda i: (0, i, 0)),
            pl.BlockSpec((1, D), lambda i: (0, 0)),
        ],
        out_specs=pl.BlockSpec((_BN, D), lambda i: (i, 0)),
        out_shape=jax.ShapeDtypeStruct((N, D), jnp.float32),
    )(p, s3, b)


def _combine_kernel(p_ref, z_ref, o_ref):
    o_ref[...] = p_ref[0] + p_ref[1] + z_ref[...]


def _tc_combine(p, z):
    return pl.pallas_call(
        _combine_kernel,
        grid=(_GRID,),
        in_specs=[
            pl.BlockSpec((NSC, _BN, D), lambda i: (0, i, 0)),
            pl.BlockSpec((_BN, D), lambda i: (i, 0)),
        ],
        out_specs=pl.BlockSpec((_BN, D), lambda i: (i, 0)),
        out_shape=jax.ShapeDtypeStruct((N, D), jnp.float32),
    )(p, z)


def kernel(x, edge_index, fc_W, fc_b, gat_W, gat_al, gat_ar, gat_b, beta):
    del beta
    e_src = edge_index[0].reshape(NT, EPT)
    e_dst = edge_index[1].reshape(NT, EPT)
    padg = jnp.broadcast_to(jnp.arange(PADE, dtype=jnp.int32), (NT, PADE))
    padp = padg + N
    srcg3 = jnp.concatenate([e_src, padg], axis=1).reshape(NT, NCH, CB)
    dstg3 = jnp.concatenate([e_dst, padg], axis=1).reshape(NT, NCH, CB)
    cb16g = srcg3 | (dstg3 << 16)
    gath_p = jnp.concatenate([e_dst, padg], axis=1)
    scat_p = jnp.concatenate([e_src, padp], axis=1)
    cb16p = (gath_p | (scat_p << 16)).reshape(NT, NCH, CB)
    zrows = jnp.zeros((N, D), jnp.float32)
    zs = jnp.zeros((SPAD,), jnp.float32)

    h = _tc_pre(x, fc_W, fc_b.reshape(1, D))
    for l in range(NUM_GNNS):
        z, el2, er2 = _tc_zelr(h, gat_W[l], gat_al[l].reshape(D, 1),
                               gat_ar[l].reshape(D, 1))
        outp, sp = _sc_edge_rows(z, el2.reshape(N), er2.reshape(N),
                                 cb16g, zrows, zs)
        s3 = sp[:, :N].reshape(NSC, N, 1)
        h = _tc_finish(outp, s3, gat_b[l].reshape(1, D))
    for _ in range(S):
        p = _sc_prop(h, cb16p, zrows)
        h = _tc_combine(p, h)
    return h


# speedup vs baseline: 1.0117x; 1.0117x over previous
import jax
import jax.numpy as jnp
from jax import lax
from jax.experimental import pallas as pl
from jax.experimental.pallas import tpu as pltpu
from jax.experimental.pallas import tpu_sc as plsc

N = 10000
E = 320000
D = 128
NUM_GNNS = 4
S = 3

NSC = 2
NSUB = 16
NT = NSC * NSUB
EPT = E // NT
CB = 128
NCH = 80
PADE = NCH * CB - EPT
NPAD = N + PADE
SPAD = 640 * NSUB

_MESH = dict(core_axis_name="c", subcore_axis_name="s",
             num_cores=NSC, num_subcores=NSUB)
_CPARAMS = pltpu.CompilerParams(needs_layout_passes=False)


def _edge_w_body(el_hbm, er_hbm, src_hbm, dst_hbm, zs_hbm,
                 w_hbm, sp_hbm,
                 sacc, srcb, dstb, elb, erb, wb, sem):
    cid = lax.axis_index("c")
    sid = lax.axis_index("s")
    wid = cid * NSUB + sid

    pltpu.sync_copy(src_hbm.at[wid], srcb)
    pltpu.sync_copy(dst_hbm.at[wid], dstb)
    pltpu.sync_copy(el_hbm, elb)
    pltpu.sync_copy(er_hbm, erb)

    @pl.when(sid == 0)
    def _():
        pltpu.sync_copy(zs_hbm, sacc)
    plsc.subcore_barrier()

    def chunk_w(c, carry):
        for v in range(CB // 16):
            ds = pl.ds(v * 16, 16)
            isrc = srcb[c, ds]
            idst = dstb[c, ds]
            a = plsc.load_gather(elb, [isrc])
            b = plsc.load_gather(erb, [idst])
            t = a + b
            wb[c, ds] = jnp.exp(jnp.where(t > 0, t, 0.2 * t))
        return carry

    lax.fori_loop(0, NCH, chunk_w, 0, unroll=4)

    zv = jnp.zeros((16,), jnp.float32)
    for p in range(EPT // 16, NCH * CB // 16):
        wb[p * 16 // CB, pl.ds((p * 16) % CB, 16)] = zv

    def chunk_s(c, carry):
        pltpu.make_async_copy(wb.at[c], sacc.at[dstb.at[c]], sem).start(
            add=True)
        return carry

    lax.fori_loop(0, NCH, chunk_s, 0)
    pltpu.sync_copy(wb, w_hbm.at[wid])

    def drain_s(c, carry):
        pltpu.make_async_copy(wb.at[0], sacc.at[dstb.at[0]], sem).wait()
        return carry

    lax.fori_loop(0, NCH, drain_s, 0)

    plsc.subcore_barrier()
    pltpu.sync_copy(sacc.at[pl.ds(sid * 640, 640)],
                    sp_hbm.at[cid, pl.ds(sid * 640, 640)])


def _scale_rows(gb, wst):
    def row(r, rc):
        wv = plsc.load_gather(wst, [jnp.zeros((16,), jnp.int32),
                                    jnp.full((16,), r, jnp.int32)])
        for k in range(D // 16):
            ds = pl.ds(k * 16, 16)
            gb[r, ds] = gb[r, ds] * wv
        return rc

    lax.fori_loop(0, CB, row, 0, unroll=16)


def _unpack_idx(cb16, c, gst, sst):
    for k in range(CB // 16):
        ds = pl.ds(k * 16, 16)
        v = cb16[c, ds]
        gst[ds] = v & jnp.int32(0xFFFF)
        sst[ds] = lax.shift_right_logical(v, 16)


def _edge_rows_body(z_hbm, w_hbm, cb16_hbm, zrows_hbm,
                    outp_hbm,
                    acc, cb16, gst0, gst1, sst0, sst1, wst0, wst1,
                    gb0, gb1, sg0, sg1, ss0, ss1, sw0, sw1):
    cid = lax.axis_index("c")
    sid = lax.axis_index("s")
    wid = cid * NSUB + sid

    pltpu.sync_copy(cb16_hbm.at[wid], cb16)

    @pl.when(sid == 0)
    def _():
        pltpu.sync_copy(zrows_hbm, acc)
    plsc.subcore_barrier()

    _unpack_idx(cb16, 0, gst0, sst0)
    pltpu.async_copy(z_hbm.at[gst0], gb0, sg0)
    pltpu.async_copy(w_hbm.at[wid, 0], wst0, sw0)

    def pair(i, carry):
        c0 = 2 * i
        c1 = c0 + 1

        @pl.when(i > 0)
        def _():
            pltpu.make_async_copy(gb1, acc.at[sst1], ss1).wait()
        _unpack_idx(cb16, c1, gst1, sst1)
        pltpu.async_copy(z_hbm.at[gst1], gb1, sg1)
        pltpu.async_copy(w_hbm.at[wid, c1], wst1, sw1)

        pltpu.make_async_copy(z_hbm.at[gst0], gb0, sg0).wait()
        pltpu.make_async_copy(w_hbm.at[wid, c0], wst0, sw0).wait()
        _scale_rows(gb0, wst0)
        pltpu.make_async_copy(gb0, acc.at[sst0], ss0).start(add=True)

        pltpu.make_async_copy(z_hbm.at[gst1], gb1, sg1).wait()
        pltpu.make_async_copy(w_hbm.at[wid, c1], wst1, sw1).wait()
        _scale_rows(gb1, wst1)

        pltpu.make_async_copy(gb0, acc.at[sst0], ss0).wait()

        @pl.when(c0 + 2 < NCH)
        def _():
            _unpack_idx(cb16, c0 + 2, gst0, sst0)
            pltpu.async_copy(z_hbm.at[gst0], gb0, sg0)
            pltpu.async_copy(w_hbm.at[wid, c0 + 2], wst0, sw0)

        pltpu.make_async_copy(gb1, acc.at[sst1], ss1).start(add=True)
        return carry

    lax.fori_loop(0, NCH // 2, pair, 0)
    pltpu.make_async_copy(gb1, acc.at[sst1], ss1).wait()

    plsc.subcore_barrier()
    base = sid * 624
    pltpu.sync_copy(acc.at[pl.ds(base, 624)],
                    outp_hbm.at[cid, pl.ds(base, 624)])

    @pl.when(sid == NSUB - 1)
    def _():
        pltpu.sync_copy(acc.at[pl.ds(16 * 624, N - 16 * 624)],
                        outp_hbm.at[cid, pl.ds(16 * 624, N - 16 * 624)])


def _prop_body(z_hbm, cb16_hbm, zrows_hbm,
               outp_hbm,
               acc, cb16, gst0, gst1, sst0, sst1, gb0, gb1,
               sg0, sg1, ss0, ss1):
    cid = lax.axis_index("c")
    sid = lax.axis_index("s")
    wid = cid * NSUB + sid

    pltpu.sync_copy(cb16_hbm.at[wid], cb16)

    @pl.when(sid == 0)
    def _():
        pltpu.sync_copy(zrows_hbm, acc.at[pl.ds(0, N)])
    plsc.subcore_barrier()

    _unpack_idx(cb16, 0, gst0, sst0)
    pltpu.async_copy(z_hbm.at[gst0], gb0, sg0)

    def pair(i, carry):
        c0 = 2 * i
        c1 = c0 + 1

        @pl.when(i > 0)
        def _():
            pltpu.make_async_copy(gb1, acc.at[sst1], ss1).wait()
        _unpack_idx(cb16, c1, gst1, sst1)
        pltpu.async_copy(z_hbm.at[gst1], gb1, sg1)

        pltpu.make_async_copy(z_hbm.at[gst0], gb0, sg0).wait()
        pltpu.make_async_copy(gb0, acc.at[sst0], ss0).start(add=True)

        pltpu.make_async_copy(z_hbm.at[gst1], gb1, sg1).wait()
        pltpu.make_async_copy(gb0, acc.at[sst0], ss0).wait()

        @pl.when(c0 + 2 < NCH)
        def _():
            _unpack_idx(cb16, c0 + 2, gst0, sst0)
            pltpu.async_copy(z_hbm.at[gst0], gb0, sg0)

        pltpu.make_async_copy(gb1, acc.at[sst1], ss1).start(add=True)
        return carry

    lax.fori_loop(0, NCH // 2, pair, 0)
    pltpu.make_async_copy(gb1, acc.at[sst1], ss1).wait()

    plsc.subcore_barrier()
    base = sid * 624
    pltpu.sync_copy(acc.at[pl.ds(base, 624)],
                    outp_hbm.at[cid, pl.ds(base, 624)])

    @pl.when(sid == NSUB - 1)
    def _():
        pltpu.sync_copy(acc.at[pl.ds(16 * 624, N - 16 * 624)],
                        outp_hbm.at[cid, pl.ds(16 * 624, N - 16 * 624)])


def _sc_edge_w(el, er, src3, dst3, zs):
    f = pl.kernel(
        _edge_w_body,
        out_type=(jax.ShapeDtypeStruct((NT, NCH, CB), jnp.float32),
                  jax.ShapeDtypeStruct((NSC, SPAD), jnp.float32)),
        mesh=plsc.VectorSubcoreMesh(**_MESH),
        compiler_params=_CPARAMS,
        scratch_types=[
            pltpu.VMEM_SHARED((SPAD,), jnp.float32),
            pltpu.VMEM((NCH, CB), jnp.int32),
            pltpu.VMEM((NCH, CB), jnp.int32),
            pltpu.VMEM((N,), jnp.float32),
            pltpu.VMEM((N,), jnp.float32),
            pltpu.VMEM((NCH, CB), jnp.float32),
            pltpu.SemaphoreType.DMA,
        ],
    )
    return f(el, er, src3, dst3, zs)


def _sc_edge_rows(z, w3, cb16g, zrows):
    f = pl.kernel(
        _edge_rows_body,
        out_type=jax.ShapeDtypeStruct((NSC, N, D), jnp.float32),
        mesh=plsc.VectorSubcoreMesh(**_MESH),
        compiler_params=_CPARAMS,
        scratch_types=[
            pltpu.VMEM_SHARED((N, D), jnp.float32),
            pltpu.VMEM((NCH, CB), jnp.int32),
            pltpu.VMEM((CB,), jnp.int32),
            pltpu.VMEM((CB,), jnp.int32),
            pltpu.VMEM((CB,), jnp.int32),
            pltpu.VMEM((CB,), jnp.int32),
            pltpu.VMEM((1, CB), jnp.float32),
            pltpu.VMEM((1, CB), jnp.float32),
            pltpu.VMEM((CB, D), jnp.float32),
            pltpu.VMEM((CB, D), jnp.float32),
            pltpu.SemaphoreType.DMA,
            pltpu.SemaphoreType.DMA,
            pltpu.SemaphoreType.DMA,
            pltpu.SemaphoreType.DMA,
            pltpu.SemaphoreType.DMA,
            pltpu.SemaphoreType.DMA,
        ],
    )
    return f(z, w3.reshape(NT, NCH, 1, CB), cb16g, zrows)


def _sc_prop(zp, cb16p, zrows):
    f = pl.kernel(
        _prop_body,
        out_type=jax.ShapeDtypeStruct((NSC, N, D), jnp.float32),
        mesh=plsc.VectorSubcoreMesh(**_MESH),
        compiler_params=_CPARAMS,
        scratch_types=[
            pltpu.VMEM_SHARED((NPAD, D), jnp.float32),
            pltpu.VMEM((NCH, CB), jnp.int32),
            pltpu.VMEM((CB,), jnp.int32),
            pltpu.VMEM((CB,), jnp.int32),
            pltpu.VMEM((CB,), jnp.int32),
            pltpu.VMEM((CB,), jnp.int32),
            pltpu.VMEM((CB, D), jnp.float32),
            pltpu.VMEM((CB, D), jnp.float32),
            pltpu.SemaphoreType.DMA,
            pltpu.SemaphoreType.DMA,
            pltpu.SemaphoreType.DMA,
            pltpu.SemaphoreType.DMA,
        ],
    )
    return f(zp, cb16p, zrows)


_GRID = 10
_BN = N // _GRID


def _pre_kernel(x_ref, w_ref, b_ref, o_ref):
    o_ref[...] = (jnp.dot(x_ref[...], w_ref[...],
                          preferred_element_type=jnp.float32) + b_ref[...])


def _tc_pre(x, fc_W, fc_b):
    return pl.pallas_call(
        _pre_kernel,
        grid=(_GRID,),
        in_specs=[
            pl.BlockSpec((_BN, D), lambda i: (i, 0)),
            pl.BlockSpec((D, D), lambda i: (0, 0)),
            pl.BlockSpec((1, D), lambda i: (0, 0)),
        ],
        out_specs=pl.BlockSpec((_BN, D), lambda i: (i, 0)),
        out_shape=jax.ShapeDtypeStruct((N, D), jnp.float32),
    )(x, fc_W, fc_b)


def _zelr_kernel(h_ref, w_ref, al_ref, ar_ref, z_ref, el_ref, er_ref):
    z = jnp.dot(h_ref[...], w_ref[...], preferred_element_type=jnp.float32)
    z_ref[...] = z
    el_ref[...] = jnp.dot(z, al_ref[...], preferred_element_type=jnp.float32)
    er_ref[...] = jnp.dot(z, ar_ref[...], preferred_element_type=jnp.float32)


def _tc_zelr(h, W, al, ar):
    return pl.pallas_call(
        _zelr_kernel,
        grid=(_GRID,),
        in_specs=[
            pl.BlockSpec((_BN, D), lambda i: (i, 0)),
            pl.BlockSpec((D, D), lambda i: (0, 0)),
            pl.BlockSpec((D, 1), lambda i: (0, 0)),
            pl.BlockSpec((D, 1), lambda i: (0, 0)),
        ],
        out_specs=[
            pl.BlockSpec((_BN, D), lambda i: (i, 0)),
            pl.BlockSpec((_BN, 1), lambda i: (i, 0)),
            pl.BlockSpec((_BN, 1), lambda i: (i, 0)),
        ],
        out_shape=[
            jax.ShapeDtypeStruct((N, D), jnp.float32),
            jax.ShapeDtypeStruct((N, 1), jnp.float32),
            jax.ShapeDtypeStruct((N, 1), jnp.float32),
        ],
    )(h, W, al, ar)


def _finish_kernel(p_ref, s_ref, b_ref, h_ref):
    ps = p_ref[0] + p_ref[1]
    ss = s_ref[0] + s_ref[1]
    t = ps / (ss + 1e-9) + b_ref[...]
    h_ref[...] = jnp.where(t > 0, t, jnp.exp(jnp.minimum(t, 0.0)) - 1.0)


def _tc_finish(p, s3, b):
    return pl.pallas_call(
        _finish_kernel,
        grid=(_GRID,),
        in_specs=[
            pl.BlockSpec((NSC, _BN, D), lambda i: (0, i, 0)),
            pl.BlockSpec((NSC, _BN, 1), lambda i: (0, i, 0)),
            pl.BlockSpec((1, D), lambda i: (0, 0)),
        ],
        out_specs=pl.BlockSpec((_BN, D), lambda i: (i, 0)),
        out_shape=jax.ShapeDtypeStruct((N, D), jnp.float32),
    )(p, s3, b)


def _combine_kernel(p_ref, z_ref, o_ref):
    o_ref[...] = p_ref[0] + p_ref[1] + z_ref[...]


def _tc_combine(p, z):
    return pl.pallas_call(
        _combine_kernel,
        grid=(_GRID,),
        in_specs=[
            pl.BlockSpec((NSC, _BN, D), lambda i: (0, i, 0)),
            pl.BlockSpec((_BN, D), lambda i: (i, 0)),
        ],
        out_specs=pl.BlockSpec((_BN, D), lambda i: (i, 0)),
        out_shape=jax.ShapeDtypeStruct((N, D), jnp.float32),
    )(p, z)


def kernel(x, edge_index, fc_W, fc_b, gat_W, gat_al, gat_ar, gat_b, beta):
    del beta
    e_src = edge_index[0].reshape(NT, EPT)
    e_dst = edge_index[1].reshape(NT, EPT)
    padg = jnp.broadcast_to(jnp.arange(PADE, dtype=jnp.int32), (NT, PADE))
    padp = padg + N
    srcg3 = jnp.concatenate([e_src, padg], axis=1).reshape(NT, NCH, CB)
    dstg3 = jnp.concatenate([e_dst, padg], axis=1).reshape(NT, NCH, CB)
    cb16g = srcg3 | (dstg3 << 16)
    gath_p = jnp.concatenate([e_dst, padg], axis=1)
    scat_p = jnp.concatenate([e_src, padp], axis=1)
    cb16p = (gath_p | (scat_p << 16)).reshape(NT, NCH, CB)
    zrows = jnp.zeros((N, D), jnp.float32)
    zs = jnp.zeros((SPAD,), jnp.float32)

    h = _tc_pre(x, fc_W, fc_b.reshape(1, D))
    for l in range(NUM_GNNS):
        z, el2, er2 = _tc_zelr(h, gat_W[l], gat_al[l].reshape(D, 1),
                               gat_ar[l].reshape(D, 1))
        w3, sp = _sc_edge_w(el2.reshape(N), er2.reshape(N), srcg3, dstg3, zs)
        outp = _sc_edge_rows(z, w3, cb16g, zrows)
        s3 = sp[:, :N].reshape(NSC, N, 1)
        h = _tc_finish(outp, s3, gat_b[l].reshape(1, D))
    for _ in range(S):
        p = _sc_prop(h, cb16p, zrows)
        h = _tc_combine(p, h)
    return h
